# Initial kernel scaffold; baseline (speedup 1.0000x reference)
import jax
import jax.numpy as jnp
from jax.experimental import pallas as pl


def kernel(input, table):
    raise NotImplementedError("write your pallas kernel here")


# speedup vs baseline: 1.1026x; 1.1026x over previous
import functools

import jax
import jax.numpy as jnp
from jax import lax
from jax.experimental import pallas as pl
from jax.experimental.pallas import tpu as pltpu
from jax.experimental.pallas import tpu_sc as plsc

_EMBED_DIM = 32
_CHUNK = 1600


@functools.partial(jax.jit, static_argnames=("n_rows",))
def _gather_rows(flat_ids, table, n_rows):
    info = plsc.get_sparse_core_info()
    nw = info.num_cores * info.num_subcores
    rows_per_w = n_rows // nw
    n_chunks = rows_per_w // _CHUNK

    mesh = plsc.VectorSubcoreMesh(core_axis_name="c", subcore_axis_name="s")

    @functools.partial(
        pl.kernel,
        mesh=mesh,
        out_type=jax.ShapeDtypeStruct((n_rows, _EMBED_DIM), jnp.float32),
        scratch_types=[
            pltpu.VMEM((_CHUNK,), jnp.int32),
            pltpu.VMEM((_CHUNK, _EMBED_DIM), jnp.float32),
            pltpu.SemaphoreType.DMA,
        ],
        compiler_params=pltpu.CompilerParams(use_tc_tiling_on_sc=False),
    )
    def k(ids_hbm, table_hbm, out_hbm, idx_v, rows_v, sem):
        wid = lax.axis_index("s") * info.num_cores + lax.axis_index("c")
        base = wid * rows_per_w

        def body(g, carry):
            off = base + g * _CHUNK
            pltpu.sync_copy(ids_hbm.at[pl.ds(off, _CHUNK)], idx_v)
            pltpu.async_copy(table_hbm.at[idx_v], rows_v, sem).wait()
            pltpu.sync_copy(rows_v, out_hbm.at[pl.ds(off, _CHUNK)])
            return carry

        lax.fori_loop(0, n_chunks, body, 0)

    return k(flat_ids, table)


def kernel(input, table):
    b, h = input.shape
    flat = _gather_rows(input.reshape(b * h), table, b * h)
    return flat.reshape(b, h, _EMBED_DIM)


# speedup vs baseline: 1.1131x; 1.0095x over previous
import functools

import jax
import jax.numpy as jnp
from jax import lax
from jax.experimental import pallas as pl
from jax.experimental.pallas import tpu as pltpu
from jax.experimental.pallas import tpu_sc as plsc

_EMBED_DIM = 32
_CHUNK = 1600
_NBUF = 2


@functools.partial(jax.jit, static_argnames=("n_rows",))
def _gather_rows(flat_ids, table, n_rows):
    info = plsc.get_sparse_core_info()
    nw = info.num_cores * info.num_subcores
    rows_per_w = n_rows // nw
    n_chunks = rows_per_w // _CHUNK

    mesh = plsc.VectorSubcoreMesh(core_axis_name="c", subcore_axis_name="s")

    @functools.partial(
        pl.kernel,
        mesh=mesh,
        out_type=jax.ShapeDtypeStruct((n_rows, _EMBED_DIM), jnp.float32),
        scratch_types=[
            pltpu.VMEM((_NBUF, _CHUNK), jnp.int32),
            pltpu.VMEM((_NBUF, _CHUNK, _EMBED_DIM), jnp.float32),
            [pltpu.SemaphoreType.DMA] * _NBUF,
            [pltpu.SemaphoreType.DMA] * _NBUF,
        ],
        compiler_params=pltpu.CompilerParams(use_tc_tiling_on_sc=False),
    )
    def k(ids_hbm, table_hbm, out_hbm, idx_v, rows_v, gsems, osems):
        wid = lax.axis_index("s") * info.num_cores + lax.axis_index("c")
        base = wid * rows_per_w

        def start_gather(g, b):
            off = base + g * _CHUNK
            pltpu.sync_copy(ids_hbm.at[pl.ds(off, _CHUNK)], idx_v.at[b])
            pltpu.async_copy(table_hbm.at[idx_v.at[b]], rows_v.at[b], gsems[b])

        def wait_gather(b):
            pltpu.make_async_copy(
                table_hbm.at[idx_v.at[b]], rows_v.at[b], gsems[b]
            ).wait()

        def start_store(g, b):
            off = base + g * _CHUNK
            pltpu.async_copy(rows_v.at[b], out_hbm.at[pl.ds(off, _CHUNK)], osems[b])

        def wait_store(g, b):
            off = base + g * _CHUNK
            pltpu.make_async_copy(
                rows_v.at[b], out_hbm.at[pl.ds(off, _CHUNK)], osems[b]
            ).wait()

        for b in range(_NBUF):
            start_gather(b, b)

        def body(go, carry):
            for b in range(_NBUF):
                g = go * _NBUF + b
                wait_gather(b)
                start_store(g, b)
                pltpu.sync_copy(
                    ids_hbm.at[pl.ds(base + (g + _NBUF) * _CHUNK, _CHUNK)],
                    idx_v.at[b],
                )
                wait_store(g, b)
                pltpu.async_copy(
                    table_hbm.at[idx_v.at[b]], rows_v.at[b], gsems[b]
                )
            return carry

        lax.fori_loop(0, n_chunks // _NBUF - 1, body, 0)

        for b in range(_NBUF):
            g = n_chunks - _NBUF + b
            wait_gather(b)
            start_store(g, b)
        for b in range(_NBUF):
            wait_store(n_chunks - _NBUF + b, b)

    return k(flat_ids, table)


def kernel(input, table):
    b, h = input.shape
    flat = _gather_rows(input.reshape(b * h), table, b * h)
    return flat.reshape(b, h, _EMBED_DIM)
